# baseline (speedup 1.0000x reference)
import jax
import jax.numpy as jnp
from jax.experimental import pallas as pl
from jax.experimental.pallas import tpu as pltpu

_RB = 128


def _body(tissue_ref, x_ref, pm_ref, reg_ref, out_ref, mask_ref, carry_ref):
    b = pl.program_id(0)
    j = pl.program_id(1)
    n_reg = reg_ref.shape[0]

    @pl.when(j == 0)
    def _first_block():
        t = tissue_ref[b, 0]
        row_ids = jax.lax.broadcasted_iota(jnp.int32, (n_reg, 1), 0)
        onehot = (row_ids == t).astype(out_ref.dtype)
        out_ref[0, 0:1, :] = jnp.sum(reg_ref[...] * onehot, axis=0,
                                     keepdims=True)
        mask_ref[0, :, 0:1] = jnp.zeros((1, 1), jnp.int32)
        mask_ref[0, :, 1:] = pm_ref[0, :, :]

    @pl.when(j > 0)
    def _later_blocks():
        out_ref[0, 0:1, :] = carry_ref[...]

    out_ref[0, 1:, :] = x_ref[0, : _RB - 1, :]
    carry_ref[...] = x_ref[0, _RB - 1 : _RB, :]


def kernel(x, tissue_vector, padding_mask, registry_tokens):
    b_sz, s_sz, d = x.shape
    nj = s_sz // _RB
    pm_i32 = padding_mask.astype(jnp.int32).reshape(b_sz, 1, s_sz)
    out, mask_i32 = pl.pallas_call(
        _body,
        grid=(b_sz, nj + 1),
        out_shape=[
            jax.ShapeDtypeStruct((b_sz, s_sz + 1, d), x.dtype),
            jax.ShapeDtypeStruct((b_sz, 1, s_sz + 1), jnp.int32),
        ],
        in_specs=[
            pl.BlockSpec(memory_space=pltpu.MemorySpace.SMEM),
            pl.BlockSpec((1, _RB, d),
                         lambda b, j: (b, jnp.minimum(j, nj - 1), 0)),
            pl.BlockSpec((1, 1, s_sz), lambda b, j: (b, 0, 0)),
            pl.BlockSpec(registry_tokens.shape, lambda b, j: (0, 0)),
        ],
        out_specs=[
            pl.BlockSpec((1, _RB, d), lambda b, j: (b, j, 0)),
            pl.BlockSpec((1, 1, s_sz + 1), lambda b, j: (b, 0, 0)),
        ],
        scratch_shapes=[pltpu.VMEM((1, d), x.dtype)],
    )(tissue_vector, x, pm_i32, registry_tokens)
    return out, mask_i32.reshape(b_sz, s_sz + 1).astype(padding_mask.dtype)


# speedup vs baseline: 1.5625x; 1.5625x over previous
import jax
import jax.numpy as jnp
from jax.experimental import pallas as pl
from jax.experimental.pallas import tpu as pltpu

_RB = 128
_BB = 8


def _body(tissue_ref, x_ref, pm_ref, reg_ref, out_ref, mask_ref, carry_ref):
    b = pl.program_id(0)
    j = pl.program_id(1)
    n_reg = reg_ref.shape[0]

    @pl.when(j == 0)
    def _first_block():
        for bb in range(_BB):
            t = tissue_ref[b * _BB + bb, 0]
            row_ids = jax.lax.broadcasted_iota(jnp.int32, (n_reg, 1), 0)
            onehot = (row_ids == t).astype(out_ref.dtype)
            out_ref[bb, 0:1, :] = jnp.sum(reg_ref[...] * onehot, axis=0,
                                          keepdims=True)
        mask_ref[:, :, 0:1] = jnp.zeros((_BB, 1, 1), jnp.int32)
        mask_ref[:, :, 1:] = pm_ref[...]

    @pl.when(j > 0)
    def _later_blocks():
        out_ref[:, 0:1, :] = carry_ref[...]

    out_ref[:, 1:, :] = x_ref[:, : _RB - 1, :]
    carry_ref[...] = x_ref[:, _RB - 1 : _RB, :]


def kernel(x, tissue_vector, padding_mask, registry_tokens):
    b_sz, s_sz, d = x.shape
    nj = s_sz // _RB
    pm_i32 = padding_mask.astype(jnp.int32).reshape(b_sz, 1, s_sz)
    out, mask_i32 = pl.pallas_call(
        _body,
        grid=(b_sz // _BB, nj + 1),
        out_shape=[
            jax.ShapeDtypeStruct((b_sz, s_sz + 1, d), x.dtype),
            jax.ShapeDtypeStruct((b_sz, 1, s_sz + 1), jnp.int32),
        ],
        in_specs=[
            pl.BlockSpec(memory_space=pltpu.MemorySpace.SMEM),
            pl.BlockSpec((_BB, _RB, d),
                         lambda b, j: (b, jnp.minimum(j, nj - 1), 0)),
            pl.BlockSpec((_BB, 1, s_sz), lambda b, j: (b, 0, 0)),
            pl.BlockSpec(registry_tokens.shape, lambda b, j: (0, 0)),
        ],
        out_specs=[
            pl.BlockSpec((_BB, _RB, d), lambda b, j: (b, j, 0)),
            pl.BlockSpec((_BB, 1, s_sz + 1), lambda b, j: (b, 0, 0)),
        ],
        scratch_shapes=[pltpu.VMEM((_BB, 1, d), x.dtype)],
    )(tissue_vector, x, pm_i32, registry_tokens)
    return out, mask_i32.reshape(b_sz, s_sz + 1).astype(padding_mask.dtype)


# speedup vs baseline: 1.5866x; 1.0155x over previous
import jax
import jax.numpy as jnp
from jax.experimental import pallas as pl
from jax.experimental.pallas import tpu as pltpu

_RB = 128
_BB = 16


def _body(tissue_ref, x_ref, pm_ref, reg_ref, out_ref, mask_ref, carry_ref):
    b = pl.program_id(0)
    j = pl.program_id(1)
    n_reg = reg_ref.shape[0]

    @pl.when(j == 0)
    def _first_block():
        for bb in range(_BB):
            t = tissue_ref[b * _BB + bb, 0]
            row_ids = jax.lax.broadcasted_iota(jnp.int32, (n_reg, 1), 0)
            onehot = (row_ids == t).astype(out_ref.dtype)
            out_ref[bb, 0:1, :] = jnp.sum(reg_ref[...] * onehot, axis=0,
                                          keepdims=True)
        mask_ref[:, :, 0:1] = jnp.zeros((_BB, 1, 1), jnp.int32)
        mask_ref[:, :, 1:] = pm_ref[...]

    @pl.when(j > 0)
    def _later_blocks():
        out_ref[:, 0:1, :] = carry_ref[...]

    out_ref[:, 1:, :] = x_ref[:, : _RB - 1, :]
    carry_ref[...] = x_ref[:, _RB - 1 : _RB, :]


def kernel(x, tissue_vector, padding_mask, registry_tokens):
    b_sz, s_sz, d = x.shape
    nj = s_sz // _RB
    pm_i32 = padding_mask.astype(jnp.int32).reshape(b_sz, 1, s_sz)
    out, mask_i32 = pl.pallas_call(
        _body,
        grid=(b_sz // _BB, nj + 1),
        out_shape=[
            jax.ShapeDtypeStruct((b_sz, s_sz + 1, d), x.dtype),
            jax.ShapeDtypeStruct((b_sz, 1, s_sz + 1), jnp.int32),
        ],
        in_specs=[
            pl.BlockSpec(memory_space=pltpu.MemorySpace.SMEM),
            pl.BlockSpec((_BB, _RB, d),
                         lambda b, j: (b, jnp.minimum(j, nj - 1), 0)),
            pl.BlockSpec((_BB, 1, s_sz), lambda b, j: (b, 0, 0)),
            pl.BlockSpec(registry_tokens.shape, lambda b, j: (0, 0)),
        ],
        out_specs=[
            pl.BlockSpec((_BB, _RB, d), lambda b, j: (b, j, 0)),
            pl.BlockSpec((_BB, 1, s_sz + 1), lambda b, j: (b, 0, 0)),
        ],
        scratch_shapes=[pltpu.VMEM((_BB, 1, d), x.dtype)],
    )(tissue_vector, x, pm_i32, registry_tokens)
    return out, mask_i32.reshape(b_sz, s_sz + 1).astype(padding_mask.dtype)
